# baseline (speedup 1.0000x reference)
import functools

import numpy as np

import jax
import jax.numpy as jnp
from jax import lax
from jax.experimental import pallas as pl
from jax.experimental.pallas import tpu as pltpu
from jax.experimental.pallas import tpu_sc as plsc

ROWS = 1024
N = 4096
M = 2 * N
L = 16
NC = 2
NS = 16
NW = NC * NS
RPW = ROWS // NW
NV = M // L
P = 2
PLANE = M // P
NCHUNK = L * P
T = PLANE // L
TSH = T.bit_length() - 1
RADIX = 256

_I32_MIN = np.int32(-(2**31))


def _to_key(v):
    b = lax.bitcast_convert_type(v, jnp.int32)
    m = lax.shift_right_arithmetic(b, 31)
    return lax.bitwise_xor(b, lax.bitwise_or(m, _I32_MIN))


def _phys(p):
    return lax.bitwise_or(
        lax.bitwise_and(p, np.int32(~(PLANE - 1))),
        lax.bitwise_or(
            lax.shift_left(lax.bitwise_and(p, T - 1), 4),
            lax.shift_right_logical(lax.bitwise_and(p, PLANE - 1), TSH)))


def _digit(k, shift):
    if shift:
        k = lax.shift_right_arithmetic(k, shift)
    return lax.bitwise_and(k, RADIX - 1)


def _sc_body(xs_hbm, xt_hbm, out_hbm,
             raw_s, raw_t, key_a, key_b, hist_n, hist_a, hist_b, accv):
    cid = lax.axis_index("c")
    sid = lax.axis_index("s")
    wid = cid * NS + sid
    lane = lax.iota(jnp.int32, L)
    ones = jnp.ones((L,), jnp.int32)
    zeros = jnp.zeros((L,), jnp.int32)
    hists = (hist_a, hist_b)

    def zero_n(i, _):
        hist_n[pl.ds(i * L, L)] = zeros
        return 0
    lax.fori_loop(0, RADIX * P, zero_n, 0, unroll=8)

    def hist_pass(inkey, shift):
        def body(t, _):
            for j in range(P):
                k = inkey[pl.ds(j * PLANE + t * L, L)]
                idx = _digit(k, shift) * NCHUNK + (j * L + lane)
                plsc.addupdate_scatter(hist_n, [idx], ones)
            return 0
        lax.fori_loop(0, T, body, 0, unroll=8)

    def prefix_pass():
        def body(d, carry):
            base = d * NCHUNK
            va = hist_n[pl.ds(base, L)]
            vb = hist_n[pl.ds(base + L, L)]
            csa = plsc.cumsum(va)
            csb = plsc.cumsum(vb)
            sa = jnp.sum(va)
            hist_a[pl.ds(d * L, L)] = csa - va + carry
            hist_b[pl.ds(d * L, L)] = csb - vb + (carry + sa)
            hist_n[pl.ds(base, L)] = zeros
            hist_n[pl.ds(base + L, L)] = zeros
            return carry + sa + jnp.sum(vb)
        lax.fori_loop(0, RADIX, body, jnp.int32(0), unroll=4)

    def permute_pass(inkey, outkey, shift, first, last):
        def body(t, _):
            for j in range(P):
                k = inkey[pl.ds(j * PLANE + t * L, L)]
                if first:
                    side01 = jnp.where(lane < (L * P // 2 - j * L),
                                       ones, zeros)
                    kout = lax.bitwise_or(
                        lax.bitwise_and(k, np.int32(~255)), side01)
                else:
                    kout = k
                idx = _digit(k, shift) * L + lane
                hj = hists[j]
                off = plsc.load_gather(hj, [idx])
                plsc.store_scatter(hj, [idx], off + 1)
                if last:
                    plsc.store_scatter(outkey, [off], kout)
                else:
                    plsc.store_scatter(outkey, [_phys(off)], kout)
            return 0
        lax.fori_loop(0, T, body, 0, unroll=4)

    def row_body(r, acc):
        row = wid * RPW + r
        pltpu.sync_copy(xs_hbm.at[row], raw_s)
        pltpu.sync_copy(xt_hbm.at[row], raw_t)

        def pre(raw, pbase):
            def body(u, _):
                v = raw[pl.ds(u * L, L)]
                p = pbase + u * L + lane
                plsc.store_scatter(key_a, [_phys(p)], _to_key(v))
                return 0
            lax.fori_loop(0, N // L, body, 0, unroll=8)
        pre(raw_s, 0)
        pre(raw_t, N)

        hist_pass(key_a, 0)
        prefix_pass()
        permute_pass(key_a, key_b, 0, True, False)
        hist_pass(key_b, 8)
        prefix_pass()
        permute_pass(key_b, key_a, 8, False, False)
        hist_pass(key_a, 16)
        prefix_pass()
        permute_pass(key_a, key_b, 16, False, False)
        hist_pass(key_b, 24)
        prefix_pass()
        permute_pass(key_b, key_a, 24, False, True)

        def walk(i, carry):
            d0, mx, mn = carry
            k = key_a[pl.ds(i * L, L)]
            s = lax.shift_left(lax.bitwise_and(k, 1), 1) - 1
            d = plsc.cumsum(s) + d0
            return (d0 + jnp.sum(s), jnp.maximum(mx, d), jnp.minimum(mn, d))
        d0, mx, mn = lax.fori_loop(
            0, NV, walk, (jnp.int32(0), zeros, zeros), unroll=4)
        dn = jnp.maximum(jnp.max(mx), -jnp.min(mn))

        f = dn.astype(jnp.float32)
        e = (f * f) * jnp.float32(-1.0 / N)
        val = jnp.float32(2.0) * jnp.exp(lax.broadcast(e, (L,)))
        return acc + jnp.where(lane < 1, val, jnp.float32(0.0))

    acc = lax.fori_loop(0, RPW, row_body, jnp.zeros((L,), jnp.float32))
    accv[...] = acc
    pltpu.sync_copy(accv, out_hbm.at[wid])


def kernel(xs, xt, alpha):
    del alpha
    mesh = plsc.VectorSubcoreMesh(
        core_axis_name="c", subcore_axis_name="s",
        num_cores=NC, num_subcores=NS)
    out = pl.kernel(
        _sc_body,
        out_type=jax.ShapeDtypeStruct((NW, L), jnp.float32),
        mesh=mesh,
        compiler_params=pltpu.CompilerParams(needs_layout_passes=False),
        scratch_types=[
            pltpu.VMEM((N,), jnp.float32),
            pltpu.VMEM((N,), jnp.float32),
            pltpu.VMEM((M,), jnp.int32),
            pltpu.VMEM((M,), jnp.int32),
            pltpu.VMEM((RADIX * NCHUNK,), jnp.int32),
            pltpu.VMEM((RADIX * L,), jnp.int32),
            pltpu.VMEM((RADIX * L,), jnp.int32),
            pltpu.VMEM((L,), jnp.float32),
        ],
    )(xs, xt)
    return jnp.sum(out) / ROWS


# speedup vs baseline: 3.0414x; 3.0414x over previous
import functools

import numpy as np

import jax
import jax.numpy as jnp
from jax import lax
from jax.experimental import pallas as pl
from jax.experimental.pallas import tpu as pltpu
from jax.experimental.pallas import tpu_sc as plsc

ROWS = 1024
N = 4096
M = 2 * N
L = 16
NC = 2
NS = 16
NW = NC * NS
RPW = ROWS // NW
NV = M // L
P = 2
PLANE = M // P
NCHUNK = L * P
T = PLANE // L
TSH = T.bit_length() - 1
RADIX = 256
G = 4

_I32_MIN = np.int32(-(2**31))


def _to_key(v):
    b = lax.bitcast_convert_type(v, jnp.int32)
    m = lax.shift_right_arithmetic(b, 31)
    return lax.bitwise_xor(b, lax.bitwise_or(m, _I32_MIN))


def _phys(p):
    return lax.bitwise_or(
        lax.bitwise_and(p, np.int32(~(PLANE - 1))),
        lax.bitwise_or(
            lax.shift_left(lax.bitwise_and(p, T - 1), 4),
            lax.shift_right_logical(lax.bitwise_and(p, PLANE - 1), TSH)))


def _digit(k, shift):
    if shift:
        k = lax.shift_right_arithmetic(k, shift)
    return lax.bitwise_and(k, RADIX - 1)


def _sc_body(xs_hbm, xt_hbm, out_hbm,
             raw_s, raw_t, key_a, key_b, hist_n, hist_a, hist_b,
             tots, sa_arr, bases_a, bases_b, accv):
    cid = lax.axis_index("c")
    sid = lax.axis_index("s")
    wid = cid * NS + sid
    lane = lax.iota(jnp.int32, L)
    ones = jnp.ones((L,), jnp.int32)
    zeros = jnp.zeros((L,), jnp.int32)
    last_lane = lane == (L - 1)
    hists = (hist_a, hist_b)
    bases = (bases_a, bases_b)

    @plsc.parallel_loop(0, RADIX * P, unroll=8)
    def _zn(i):
        hist_n[pl.ds(i * L, L)] = zeros

    def hist_pass(inkey, shift):
        @plsc.parallel_loop(0, T, unroll=8)
        def _hist_body(t):
            for j in range(P):
                k = inkey[pl.ds(j * PLANE + t * L, L)]
                idx = _digit(k, shift) * NCHUNK + (j * L + lane)
                plsc.addupdate_scatter(hist_n, [idx], ones)

    def prefix_pass():
        @plsc.parallel_loop(0, RADIX, unroll=4)
        def _p1(d):
            base = d * NCHUNK
            va = hist_n[pl.ds(base, L)]
            vb = hist_n[pl.ds(base + L, L)]
            csa = plsc.cumsum(va)
            csb = plsc.cumsum(vb)
            dvec = d + lane * 0
            hist_a[pl.ds(d * L, L)] = csa - va
            hist_b[pl.ds(d * L, L)] = csb - vb
            plsc.store_scatter(tots, [dvec], csa + csb, mask=last_lane)
            plsc.store_scatter(sa_arr, [dvec], csa, mask=last_lane)
            hist_n[pl.ds(base, L)] = zeros
            hist_n[pl.ds(base + L, L)] = zeros

        def _p2(g, carry):
            v = tots[pl.ds(g * L, L)]
            cs = plsc.cumsum(v)
            ex = cs - v + carry
            bases_a[pl.ds(g * L, L)] = ex
            bases_b[pl.ds(g * L, L)] = ex + sa_arr[pl.ds(g * L, L)]
            return carry + jnp.sum(v)
        lax.fori_loop(0, RADIX // L, _p2, jnp.int32(0), unroll=4)

    def permute_pass(inkey, outkey, shift, first, last):
        def body(q, _):
            ks = [[inkey[pl.ds(j * PLANE + (q * G + g) * L, L)]
                   for g in range(G)] for j in range(P)]
            ds_ = [[_digit(ks[j][g], shift) for g in range(G)]
                   for j in range(P)]
            idxs = [[ds_[j][g] * L + lane for g in range(G)]
                    for j in range(P)]
            offs = [[plsc.load_gather(hists[j], [idxs[j][g]])
                     for g in range(G)] for j in range(P)]
            bs = [[plsc.load_gather(bases[j], [ds_[j][g]])
                   for g in range(G)] for j in range(P)]
            for j in range(P):
                for g in range(G):
                    plsc.addupdate_scatter(hists[j], [idxs[j][g]], ones)
            for j in range(P):
                for g in range(G):
                    o = offs[j][g] + bs[j][g]
                    for h in range(g):
                        o = o + jnp.where(ds_[j][h] == ds_[j][g], 1, 0)
                    k = ks[j][g]
                    if first:
                        side01 = jnp.where(lane < (L * P // 2 - j * L),
                                           ones, zeros)
                        k = lax.bitwise_or(
                            lax.bitwise_and(k, np.int32(~255)), side01)
                    if last:
                        plsc.store_scatter(outkey, [o], k)
                    else:
                        plsc.store_scatter(outkey, [_phys(o)], k)
            return 0
        lax.fori_loop(0, T // G, body, 0)

    def row_body(r, acc):
        row = wid * RPW + r
        pltpu.sync_copy(xs_hbm.at[row], raw_s)
        pltpu.sync_copy(xt_hbm.at[row], raw_t)

        def pre(raw, pbase):
            @plsc.parallel_loop(0, N // L, unroll=8)
            def _pre_body(u):
                v = raw[pl.ds(u * L, L)]
                p = pbase + u * L + lane
                plsc.store_scatter(key_a, [_phys(p)], _to_key(v))
        pre(raw_s, 0)
        pre(raw_t, N)

        hist_pass(key_a, 0)
        prefix_pass()
        permute_pass(key_a, key_b, 0, True, False)
        hist_pass(key_b, 8)
        prefix_pass()
        permute_pass(key_b, key_a, 8, False, False)
        hist_pass(key_a, 16)
        prefix_pass()
        permute_pass(key_a, key_b, 16, False, False)
        hist_pass(key_b, 24)
        prefix_pass()
        permute_pass(key_b, key_a, 24, False, True)

        def walk(i, carry):
            d0, mx, mn = carry
            k = key_a[pl.ds(i * L, L)]
            s = lax.shift_left(lax.bitwise_and(k, 1), 1) - 1
            d = plsc.cumsum(s) + d0
            return (d0 + jnp.sum(s), jnp.maximum(mx, d), jnp.minimum(mn, d))
        d0, mx, mn = lax.fori_loop(
            0, NV, walk, (jnp.int32(0), zeros, zeros), unroll=4)
        dn = jnp.maximum(jnp.max(mx), -jnp.min(mn))

        f = dn.astype(jnp.float32)
        e = (f * f) * jnp.float32(-1.0 / N)
        val = jnp.float32(2.0) * jnp.exp(lax.broadcast(e, (L,)))
        return acc + jnp.where(lane < 1, val, jnp.float32(0.0))

    acc = lax.fori_loop(0, RPW, row_body, jnp.zeros((L,), jnp.float32))
    accv[...] = acc
    pltpu.sync_copy(accv, out_hbm.at[wid])


def kernel(xs, xt, alpha):
    del alpha
    mesh = plsc.VectorSubcoreMesh(
        core_axis_name="c", subcore_axis_name="s",
        num_cores=NC, num_subcores=NS)
    out = pl.kernel(
        _sc_body,
        out_type=jax.ShapeDtypeStruct((NW, L), jnp.float32),
        mesh=mesh,
        compiler_params=pltpu.CompilerParams(needs_layout_passes=False),
        scratch_types=[
            pltpu.VMEM((N,), jnp.float32),
            pltpu.VMEM((N,), jnp.float32),
            pltpu.VMEM((M,), jnp.int32),
            pltpu.VMEM((M,), jnp.int32),
            pltpu.VMEM((RADIX * NCHUNK,), jnp.int32),
            pltpu.VMEM((RADIX * L,), jnp.int32),
            pltpu.VMEM((RADIX * L,), jnp.int32),
            pltpu.VMEM((RADIX,), jnp.int32),
            pltpu.VMEM((RADIX,), jnp.int32),
            pltpu.VMEM((RADIX,), jnp.int32),
            pltpu.VMEM((RADIX,), jnp.int32),
            pltpu.VMEM((L,), jnp.float32),
        ],
    )(xs, xt)
    return jnp.sum(out) / ROWS


# speedup vs baseline: 3.2161x; 1.0575x over previous
import functools

import numpy as np

import jax
import jax.numpy as jnp
from jax import lax
from jax.experimental import pallas as pl
from jax.experimental.pallas import tpu as pltpu
from jax.experimental.pallas import tpu_sc as plsc

ROWS = 1024
N = 4096
M = 2 * N
L = 16
NC = 2
NS = 16
NW = NC * NS
RPW = ROWS // NW
NV = M // L
P = 2
PLANE = M // P
NCHUNK = L * P
T = PLANE // L
TSH = T.bit_length() - 1
RADIX = 256
G = 4

_I32_MIN = np.int32(-(2**31))


def _to_key(v):
    b = lax.bitcast_convert_type(v, jnp.int32)
    m = lax.shift_right_arithmetic(b, 31)
    return lax.bitwise_xor(b, lax.bitwise_or(m, _I32_MIN))


def _phys(p):
    return lax.bitwise_or(
        lax.bitwise_and(p, np.int32(~(PLANE - 1))),
        lax.bitwise_or(
            lax.shift_left(lax.bitwise_and(p, T - 1), 4),
            lax.shift_right_logical(lax.bitwise_and(p, PLANE - 1), TSH)))


def _digit(k, shift):
    if shift:
        k = lax.shift_right_arithmetic(k, shift)
    return lax.bitwise_and(k, RADIX - 1)


def _sc_body(xs_hbm, xt_hbm, out_hbm,
             raw0, raw1, sem0, sem1, key_a, key_b, hist_n, hist_a, hist_b,
             tots, sa_arr, bases_a, bases_b, accv):
    cid = lax.axis_index("c")
    sid = lax.axis_index("s")
    wid = cid * NS + sid
    lane = lax.iota(jnp.int32, L)
    ones = jnp.ones((L,), jnp.int32)
    zeros = jnp.zeros((L,), jnp.int32)
    last_lane = lane == (L - 1)
    hists = (hist_a, hist_b)
    bases = (bases_a, bases_b)

    @plsc.parallel_loop(0, RADIX * P, unroll=8)
    def _zn(i):
        hist_n[pl.ds(i * L, L)] = zeros

    def hist_pass(inkey, shift):
        @plsc.parallel_loop(0, T, unroll=8)
        def _hist_body(t):
            for j in range(P):
                k = inkey[pl.ds(j * PLANE + t * L, L)]
                idx = _digit(k, shift) * NCHUNK + (j * L + lane)
                plsc.addupdate_scatter(hist_n, [idx], ones)

    def prefix_pass():
        @plsc.parallel_loop(0, RADIX, unroll=4)
        def _p1(d):
            base = d * NCHUNK
            va = hist_n[pl.ds(base, L)]
            vb = hist_n[pl.ds(base + L, L)]
            csa = plsc.cumsum(va)
            csb = plsc.cumsum(vb)
            dvec = d + lane * 0
            hist_a[pl.ds(d * L, L)] = csa - va
            hist_b[pl.ds(d * L, L)] = csb - vb
            plsc.store_scatter(tots, [dvec], csa + csb, mask=last_lane)
            plsc.store_scatter(sa_arr, [dvec], csa, mask=last_lane)
            hist_n[pl.ds(base, L)] = zeros
            hist_n[pl.ds(base + L, L)] = zeros

        def _p2(g, carry):
            v = tots[pl.ds(g * L, L)]
            cs = plsc.cumsum(v)
            ex = cs - v + carry
            bases_a[pl.ds(g * L, L)] = ex
            bases_b[pl.ds(g * L, L)] = ex + sa_arr[pl.ds(g * L, L)]
            return carry + jnp.sum(v)
        lax.fori_loop(0, RADIX // L, _p2, jnp.int32(0), unroll=4)

    def permute_pass(inkey, outkey, shift, first, last):
        def body(q, _):
            ks = [[inkey[pl.ds(j * PLANE + (q * G + g) * L, L)]
                   for g in range(G)] for j in range(P)]
            ds_ = [[_digit(ks[j][g], shift) for g in range(G)]
                   for j in range(P)]
            idxs = [[ds_[j][g] * L + lane for g in range(G)]
                    for j in range(P)]
            offs = [[plsc.load_gather(hists[j], [idxs[j][g]])
                     for g in range(G)] for j in range(P)]
            bs = [[plsc.load_gather(bases[j], [ds_[j][g]])
                   for g in range(G)] for j in range(P)]
            for j in range(P):
                for g in range(G):
                    plsc.addupdate_scatter(hists[j], [idxs[j][g]], ones)
            for j in range(P):
                for g in range(G):
                    o = offs[j][g] + bs[j][g]
                    for h in range(g):
                        o = o + jnp.where(ds_[j][h] == ds_[j][g], 1, 0)
                    k = ks[j][g]
                    if first:
                        side01 = jnp.where(lane < (L * P // 2 - j * L),
                                           ones, zeros)
                        k = lax.bitwise_or(
                            lax.bitwise_and(k, np.int32(~255)), side01)
                    if last:
                        plsc.store_scatter(outkey, [o], k)
                    else:
                        plsc.store_scatter(outkey, [_phys(o)], k)
            return 0
        lax.fori_loop(0, T // G, body, 0)

    def fetch(row, raw, sem):
        pltpu.async_copy(xs_hbm.at[row], raw.at[pl.ds(0, N)], sem)
        pltpu.async_copy(xt_hbm.at[row], raw.at[pl.ds(N, N)], sem)

    def fetch_wait(raw, sem):
        pltpu.make_async_copy(xs_hbm.at[0], raw.at[pl.ds(0, N)], sem).wait()
        pltpu.make_async_copy(xt_hbm.at[0], raw.at[pl.ds(N, N)], sem).wait()

    def row_body(raw, sem, next_row, nraw, nsem, acc):
        fetch_wait(raw, sem)
        fetch(next_row, nraw, nsem)

        def pre(pbase):
            @plsc.parallel_loop(0, N // L, unroll=8)
            def _pre_body(u):
                v = raw[pl.ds(pbase + u * L, L)]
                p = pbase + u * L + lane
                plsc.store_scatter(key_a, [_phys(p)], _to_key(v))
        pre(0)
        pre(N)

        hist_pass(key_a, 0)
        prefix_pass()
        permute_pass(key_a, key_b, 0, True, False)
        hist_pass(key_b, 8)
        prefix_pass()
        permute_pass(key_b, key_a, 8, False, False)
        hist_pass(key_a, 16)
        prefix_pass()
        permute_pass(key_a, key_b, 16, False, False)
        hist_pass(key_b, 24)
        prefix_pass()
        permute_pass(key_b, key_a, 24, False, True)

        def walk(i, carry):
            d0, mx, mn = carry
            k = key_a[pl.ds(i * L, L)]
            s = lax.shift_left(lax.bitwise_and(k, 1), 1) - 1
            d = plsc.cumsum(s) + d0
            return (d0 + jnp.sum(s), jnp.maximum(mx, d), jnp.minimum(mn, d))
        d0, mx, mn = lax.fori_loop(
            0, NV, walk, (jnp.int32(0), zeros, zeros), unroll=4)
        dn = jnp.maximum(jnp.max(mx), -jnp.min(mn))

        f = dn.astype(jnp.float32)
        e = (f * f) * jnp.float32(-1.0 / N)
        val = jnp.float32(2.0) * jnp.exp(lax.broadcast(e, (L,)))
        return acc + jnp.where(lane < 1, val, jnp.float32(0.0))

    base = wid * RPW
    fetch(base, raw0, sem0)

    def pair_body(q, acc):
        acc = row_body(raw0, sem0, base + 2 * q + 1, raw1, sem1, acc)
        acc = row_body(raw1, sem1,
                       jnp.minimum(base + 2 * q + 2, ROWS - 1),
                       raw0, sem0, acc)
        return acc
    acc = lax.fori_loop(0, RPW // 2, pair_body,
                        jnp.zeros((L,), jnp.float32))
    fetch_wait(raw0, sem0)
    accv[...] = acc
    pltpu.sync_copy(accv, out_hbm.at[wid])


def kernel(xs, xt, alpha):
    del alpha
    mesh = plsc.VectorSubcoreMesh(
        core_axis_name="c", subcore_axis_name="s",
        num_cores=NC, num_subcores=NS)
    out = pl.kernel(
        _sc_body,
        out_type=jax.ShapeDtypeStruct((NW, L), jnp.float32),
        mesh=mesh,
        compiler_params=pltpu.CompilerParams(needs_layout_passes=False),
        scratch_types=[
            pltpu.VMEM((M,), jnp.float32),
            pltpu.VMEM((M,), jnp.float32),
            pltpu.SemaphoreType.DMA,
            pltpu.SemaphoreType.DMA,
            pltpu.VMEM((M,), jnp.int32),
            pltpu.VMEM((M,), jnp.int32),
            pltpu.VMEM((RADIX * NCHUNK,), jnp.int32),
            pltpu.VMEM((RADIX * L,), jnp.int32),
            pltpu.VMEM((RADIX * L,), jnp.int32),
            pltpu.VMEM((RADIX,), jnp.int32),
            pltpu.VMEM((RADIX,), jnp.int32),
            pltpu.VMEM((RADIX,), jnp.int32),
            pltpu.VMEM((RADIX,), jnp.int32),
            pltpu.VMEM((L,), jnp.float32),
        ],
    )(xs, xt)
    return jnp.sum(out) / ROWS


# speedup vs baseline: 3.2542x; 1.0118x over previous
import functools

import numpy as np

import jax
import jax.numpy as jnp
from jax import lax
from jax.experimental import pallas as pl
from jax.experimental.pallas import tpu as pltpu
from jax.experimental.pallas import tpu_sc as plsc

ROWS = 1024
N = 4096
M = 2 * N
L = 16
NC = 2
NS = 16
NW = NC * NS
RPW = ROWS // NW
NV = M // L
P = 2
PLANE = M // P
NCHUNK = L * P
T = PLANE // L
TSH = T.bit_length() - 1
RADIX = 256
G = 4

_I32_MIN = np.int32(-(2**31))


def _to_key(v):
    b = lax.bitcast_convert_type(v, jnp.int32)
    m = lax.shift_right_arithmetic(b, 31)
    return lax.bitwise_xor(b, lax.bitwise_or(m, _I32_MIN))


def _phys(p):
    return lax.bitwise_or(
        lax.bitwise_and(p, np.int32(~(PLANE - 1))),
        lax.bitwise_or(
            lax.shift_left(lax.bitwise_and(p, T - 1), 4),
            lax.shift_right_logical(lax.bitwise_and(p, PLANE - 1), TSH)))


def _digit(k, shift):
    if shift:
        k = lax.shift_right_arithmetic(k, shift)
    return lax.bitwise_and(k, RADIX - 1)


def _sc_body(xs_hbm, xt_hbm, out_hbm,
             raw0, raw1, sem0, sem1, key_a, key_b, hist_n, hist_a, hist_b,
             tots, sa_arr, bases_a, bases_b, accv):
    cid = lax.axis_index("c")
    sid = lax.axis_index("s")
    wid = cid * NS + sid
    lane = lax.iota(jnp.int32, L)
    ones = jnp.ones((L,), jnp.int32)
    zeros = jnp.zeros((L,), jnp.int32)
    last_lane = lane == (L - 1)
    hists = (hist_a, hist_b)
    bases = (bases_a, bases_b)

    @plsc.parallel_loop(0, RADIX * P, unroll=8)
    def _zn(i):
        hist_n[pl.ds(i * L, L)] = zeros

    lane16 = lax.shift_left(lane, 4)

    def _sphys(p):
        return ((p & ~(PLANE - 1)) | ((p & (T - 1)) << 4)
                | ((p & (PLANE - 1)) >> TSH))

    def hist_pass(inkey, shift):
        @plsc.parallel_loop(0, T, unroll=8)
        def _hist_body(t):
            for j in range(P):
                k = inkey[pl.ds(j * PLANE + t * L, L)]
                idx = _digit(k, shift) * NCHUNK + (j * L + lane)
                plsc.addupdate_scatter(hist_n, [idx], ones)

    def prefix_pass():
        @plsc.parallel_loop(0, RADIX, unroll=4)
        def _p1(d):
            base = d * NCHUNK
            va = hist_n[pl.ds(base, L)]
            vb = hist_n[pl.ds(base + L, L)]
            csa = plsc.cumsum(va)
            csb = plsc.cumsum(vb)
            dvec = d + lane * 0
            hist_a[pl.ds(d * L, L)] = csa - va
            hist_b[pl.ds(d * L, L)] = csb - vb
            plsc.store_scatter(tots, [dvec], csa + csb, mask=last_lane)
            plsc.store_scatter(sa_arr, [dvec], csa, mask=last_lane)
            hist_n[pl.ds(base, L)] = zeros
            hist_n[pl.ds(base + L, L)] = zeros

        def _p2(g, carry):
            v = tots[pl.ds(g * L, L)]
            cs = plsc.cumsum(v)
            ex = cs - v + carry
            bases_a[pl.ds(g * L, L)] = ex
            bases_b[pl.ds(g * L, L)] = ex + sa_arr[pl.ds(g * L, L)]
            return carry + jnp.sum(v)
        lax.fori_loop(0, RADIX // L, _p2, jnp.int32(0), unroll=4)

    def permute_pass(inkey, outkey, shift, first, last):
        def body(q, _):
            ks = [[inkey[pl.ds(j * PLANE + (q * G + g) * L, L)]
                   for g in range(G)] for j in range(P)]
            ds_ = [[_digit(ks[j][g], shift) for g in range(G)]
                   for j in range(P)]
            idxs = [[ds_[j][g] * L + lane for g in range(G)]
                    for j in range(P)]
            offs = [[plsc.load_gather(hists[j], [idxs[j][g]])
                     for g in range(G)] for j in range(P)]
            bs = [[plsc.load_gather(bases[j], [ds_[j][g]])
                   for g in range(G)] for j in range(P)]
            for j in range(P):
                for g in range(G):
                    plsc.addupdate_scatter(hists[j], [idxs[j][g]], ones)
            for j in range(P):
                for g in range(G):
                    o = offs[j][g] + bs[j][g]
                    for h in range(g):
                        o = o + jnp.where(ds_[j][h] == ds_[j][g], 1, 0)
                    k = ks[j][g]
                    if first:
                        side01 = jnp.where(lane < (L * P // 2 - j * L),
                                           ones, zeros)
                        k = lax.bitwise_or(
                            lax.bitwise_and(k, np.int32(~255)), side01)
                    if last:
                        plsc.store_scatter(outkey, [o], k)
                    else:
                        plsc.store_scatter(outkey, [_phys(o)], k)
            return 0
        lax.fori_loop(0, T // G, body, 0)

    def fetch(row, raw, sem):
        pltpu.async_copy(xs_hbm.at[row], raw.at[pl.ds(0, N)], sem)
        pltpu.async_copy(xt_hbm.at[row], raw.at[pl.ds(N, N)], sem)

    def fetch_wait(raw, sem):
        pltpu.make_async_copy(xs_hbm.at[0], raw.at[pl.ds(0, N)], sem).wait()
        pltpu.make_async_copy(xt_hbm.at[0], raw.at[pl.ds(N, N)], sem).wait()

    def row_body(raw, sem, next_row, nraw, nsem, acc):
        fetch_wait(raw, sem)
        fetch(next_row, nraw, nsem)

        def pre(pbase):
            @plsc.parallel_loop(0, N // L, unroll=8)
            def _pre_body(u):
                v = raw[pl.ds(pbase + u * L, L)]
                phys = _sphys(pbase + u * L) + lane16
                plsc.store_scatter(key_a, [phys], _to_key(v))
        pre(0)
        pre(N)

        hist_pass(key_a, 0)
        prefix_pass()
        permute_pass(key_a, key_b, 0, True, False)
        hist_pass(key_b, 8)
        prefix_pass()
        permute_pass(key_b, key_a, 8, False, False)
        hist_pass(key_a, 16)
        prefix_pass()
        permute_pass(key_a, key_b, 16, False, False)
        hist_pass(key_b, 24)
        prefix_pass()
        permute_pass(key_b, key_a, 24, False, True)

        def walk(i, carry):
            d0, mx, mn = carry
            k = key_a[pl.ds(i * L, L)]
            s = lax.shift_left(lax.bitwise_and(k, 1), 1) - 1
            d = plsc.cumsum(s) + d0
            return (d0 + jnp.sum(s), jnp.maximum(mx, d), jnp.minimum(mn, d))
        d0, mx, mn = lax.fori_loop(
            0, NV, walk, (jnp.int32(0), zeros, zeros), unroll=4)
        dn = jnp.maximum(jnp.max(mx), -jnp.min(mn))

        f = dn.astype(jnp.float32)
        e = (f * f) * jnp.float32(-1.0 / N)
        val = jnp.float32(2.0) * jnp.exp(lax.broadcast(e, (L,)))
        return acc + jnp.where(lane < 1, val, jnp.float32(0.0))

    base = wid * RPW
    fetch(base, raw0, sem0)

    def pair_body(q, acc):
        acc = row_body(raw0, sem0, base + 2 * q + 1, raw1, sem1, acc)
        acc = row_body(raw1, sem1,
                       jnp.minimum(base + 2 * q + 2, ROWS - 1),
                       raw0, sem0, acc)
        return acc
    acc = lax.fori_loop(0, RPW // 2, pair_body,
                        jnp.zeros((L,), jnp.float32))
    fetch_wait(raw0, sem0)
    accv[...] = acc
    pltpu.sync_copy(accv, out_hbm.at[wid])


def kernel(xs, xt, alpha):
    del alpha
    mesh = plsc.VectorSubcoreMesh(
        core_axis_name="c", subcore_axis_name="s",
        num_cores=NC, num_subcores=NS)
    out = pl.kernel(
        _sc_body,
        out_type=jax.ShapeDtypeStruct((NW, L), jnp.float32),
        mesh=mesh,
        compiler_params=pltpu.CompilerParams(needs_layout_passes=False),
        scratch_types=[
            pltpu.VMEM((M,), jnp.float32),
            pltpu.VMEM((M,), jnp.float32),
            pltpu.SemaphoreType.DMA,
            pltpu.SemaphoreType.DMA,
            pltpu.VMEM((M,), jnp.int32),
            pltpu.VMEM((M,), jnp.int32),
            pltpu.VMEM((RADIX * NCHUNK,), jnp.int32),
            pltpu.VMEM((RADIX * L,), jnp.int32),
            pltpu.VMEM((RADIX * L,), jnp.int32),
            pltpu.VMEM((RADIX,), jnp.int32),
            pltpu.VMEM((RADIX,), jnp.int32),
            pltpu.VMEM((RADIX,), jnp.int32),
            pltpu.VMEM((RADIX,), jnp.int32),
            pltpu.VMEM((L,), jnp.float32),
        ],
    )(xs, xt)
    return jnp.sum(out) / ROWS
